# baseline (speedup 1.0000x reference)
import functools

import jax
import jax.numpy as jnp
from jax import lax
from jax.experimental import pallas as pl
from jax.experimental.pallas import tpu as pltpu
from jax.experimental.pallas import tpu_sc as plsc

N = 10000
E = 320000
D = 128
D_HID = 256

NC, NS, L = 2, 16, 16
NW = NC * NS
EPW = E // NW
CB = 4000
CBW = 2000
FPT = D // NW

_MESH = plsc.VectorSubcoreMesh(core_axis_name="c", subcore_axis_name="s",
                               num_cores=NC, num_subcores=NS)
_SC_PARAMS = pltpu.CompilerParams(needs_layout_passes=False)


def _wid():
    return lax.axis_index("s") * NC + lax.axis_index("c")


@functools.partial(
    pl.kernel,
    out_type=jax.ShapeDtypeStruct((NW, N), jnp.float32),
    mesh=_MESH,
    compiler_params=_SC_PARAMS,
    scratch_types=[
        pltpu.VMEM((N,), jnp.float32),
        pltpu.VMEM((EPW,), jnp.int32),
        pltpu.VMEM((EPW,), jnp.float32),
        pltpu.SemaphoreType.DMA,
    ],
)
def _deg_partials(col_hbm, ew_hbm, out_hbm, acc, colbuf, ewbuf, sem):
    wid = _wid()
    base = wid * EPW
    pltpu.async_copy(col_hbm.at[pl.ds(base, EPW)], colbuf, sem)
    pltpu.async_copy(ew_hbm.at[pl.ds(base, EPW)], ewbuf, sem)

    @plsc.parallel_loop(0, N // L, unroll=5)
    def zero_body(i):
        acc[pl.ds(i * L, L)] = jnp.zeros((L,), jnp.float32)

    pltpu.make_async_copy(col_hbm.at[pl.ds(0, EPW)], colbuf, sem).wait()
    pltpu.make_async_copy(ew_hbm.at[pl.ds(0, EPW)], ewbuf, sem).wait()

    @plsc.parallel_loop(0, EPW // L, unroll=5)
    def body(i):
        idx = colbuf[pl.ds(i * L, L)]
        w = ewbuf[pl.ds(i * L, L)]
        plsc.addupdate_scatter(acc, [idx], w)

    pltpu.sync_copy(acc, out_hbm.at[wid])


@functools.partial(
    pl.kernel,
    out_type=(jax.ShapeDtypeStruct((E,), jnp.float32),
              jax.ShapeDtypeStruct((E,), jnp.int32)),
    mesh=_MESH,
    compiler_params=_SC_PARAMS,
    scratch_types=[
        pltpu.VMEM((N,), jnp.float32),
        pltpu.VMEM((EPW,), jnp.int32),
        pltpu.VMEM((EPW,), jnp.int32),
        pltpu.VMEM((EPW,), jnp.float32),
        pltpu.VMEM((EPW,), jnp.float32),
        pltpu.VMEM((EPW,), jnp.int32),
        pltpu.SemaphoreType.DMA,
    ],
)
def _edge_norm(row_hbm, col_hbm, ew_hbm, dinv_hbm, out_hbm, pidx_hbm,
               dinvbuf, rowbuf, colbuf, ewbuf, normbuf, pidxbuf, sem):
    wid = _wid()
    base = wid * EPW
    pltpu.async_copy(row_hbm.at[pl.ds(base, EPW)], rowbuf, sem)
    pltpu.async_copy(col_hbm.at[pl.ds(base, EPW)], colbuf, sem)
    pltpu.async_copy(ew_hbm.at[pl.ds(base, EPW)], ewbuf, sem)
    pltpu.sync_copy(dinv_hbm, dinvbuf)
    pltpu.make_async_copy(row_hbm.at[pl.ds(0, EPW)], rowbuf, sem).wait()
    pltpu.make_async_copy(col_hbm.at[pl.ds(0, EPW)], colbuf, sem).wait()
    pltpu.make_async_copy(ew_hbm.at[pl.ds(0, EPW)], ewbuf, sem).wait()

    @plsc.parallel_loop(0, EPW // L, unroll=5)
    def body(i):
        s = pl.ds(i * L, L)
        r = rowbuf[s]
        cc = colbuf[s]
        w = ewbuf[s]
        dr = plsc.load_gather(dinvbuf, [r])
        dc = plsc.load_gather(dinvbuf, [cc])
        normbuf[s] = dr * w * dc
        pidxbuf[s] = jnp.bitwise_or(jnp.left_shift(r, 16), cc)

    pltpu.sync_copy(normbuf, out_hbm.at[pl.ds(base, EPW)])
    pltpu.sync_copy(pidxbuf, pidx_hbm.at[pl.ds(base, EPW)])


@functools.partial(
    pl.kernel,
    out_type=jax.ShapeDtypeStruct((D, N), jnp.float32),
    mesh=_MESH,
    compiler_params=_SC_PARAMS,
    scratch_types=[
        pltpu.VMEM((FPT, N), jnp.float32),
        pltpu.VMEM((FPT, N), jnp.float32),
        pltpu.VMEM((N,), jnp.float32),
        pltpu.VMEM((CB,), jnp.int32),
        pltpu.VMEM((CB,), jnp.int32),
        pltpu.VMEM((CB,), jnp.float32),
        pltpu.VMEM((CB,), jnp.float32),
        pltpu.SemaphoreType.DMA,
        pltpu.SemaphoreType.DMA,
    ],
)
def _agg_t(vt_hbm, pidx_hbm, norm_hbm, d2_hbm, out_hbm,
           vt, acc, d2buf, pidxbuf0, pidxbuf1, normbuf0, normbuf1,
           sem0, sem1):
    wid = _wid()
    f0 = wid * FPT
    NB = E // CB
    sems = (sem0, sem1)
    pidxbufs = (pidxbuf0, pidxbuf1)
    normbufs = (normbuf0, normbuf1)

    def issue(blk, slot):
        off = pl.multiple_of(blk * CB, 8)
        pltpu.async_copy(pidx_hbm.at[pl.ds(off, CB)], pidxbufs[slot], sems[slot])
        pltpu.async_copy(norm_hbm.at[pl.ds(off, CB)], normbufs[slot], sems[slot])

    def drain(slot):
        pltpu.make_async_copy(pidx_hbm.at[pl.ds(0, CB)], pidxbufs[slot],
                              sems[slot]).wait()
        pltpu.make_async_copy(norm_hbm.at[pl.ds(0, CB)], normbufs[slot],
                              sems[slot]).wait()

    issue(0, 0)
    issue(1, 1)

    pltpu.sync_copy(vt_hbm.at[pl.ds(f0, FPT)], vt)
    pltpu.sync_copy(d2_hbm, d2buf)

    for f in range(FPT):
        @plsc.parallel_loop(0, N // L, unroll=5)
        def init_body(i, f=f):
            s = pl.ds(i * L, L)
            acc[f, s] = vt[f, s] * d2buf[s]

    fidx = [jnp.full((L,), f, jnp.int32) for f in range(FPT)]

    def blk_body(k, c):
        for slot in range(2):
            blk = 2 * k + slot
            drain(slot)

            @plsc.parallel_loop(0, CB // L, unroll=10)
            def body(i, slot=slot):
                s = pl.ds(i * L, L)
                pv = pidxbufs[slot][s]
                r = jnp.right_shift(pv, 16)
                cc = jnp.bitwise_and(pv, jnp.int32(0xFFFF))
                nv = normbufs[slot][s]
                for f in range(FPT):
                    g = plsc.load_gather(vt, [fidx[f], r])
                    plsc.addupdate_scatter(acc, [fidx[f], cc], g * nv)

            @pl.when(blk + 2 < NB)
            def _(blk=blk, slot=slot):
                issue(blk + 2, slot)
        return c
    lax.fori_loop(0, NB // 2, blk_body, 0)

    pltpu.sync_copy(acc, out_hbm.at[pl.ds(f0, FPT)])


def _deg_finish_body(part_ref, x_ref, dinv_ref, d2_ref, xt_ref):
    deg = jnp.sum(part_ref[...], axis=0, keepdims=True) + 1.0
    dinv = jnp.where(deg > 0, lax.rsqrt(deg), 0.0)
    dinv_ref[...] = dinv
    d2_ref[...] = dinv * dinv
    xt_ref[...] = x_ref[...].T


def _deg_finish(part, x):
    return pl.pallas_call(
        _deg_finish_body,
        out_shape=(jax.ShapeDtypeStruct((1, N), jnp.float32),
                   jax.ShapeDtypeStruct((1, N), jnp.float32),
                   jax.ShapeDtypeStruct((D, N), jnp.float32)),
    )(part, x)


def _mm1_body(s1t_ref, w1_ref, b1_ref, w2_ref, xemb_ref, gt_ref):
    xe = lax.dot_general(s1t_ref[...], w1_ref[...],
                         (((0,), (0,)), ((), ())),
                         preferred_element_type=jnp.float32) + b1_ref[...]
    xemb_ref[...] = xe
    g = jnp.dot(jnp.maximum(xe, 0.0), w2_ref[...],
                preferred_element_type=jnp.float32)
    gt_ref[...] = g.T


def _mm1(s1t, W1, b1, W2):
    return pl.pallas_call(
        _mm1_body,
        out_shape=(jax.ShapeDtypeStruct((N, D_HID), jnp.float32),
                   jax.ShapeDtypeStruct((D, N), jnp.float32)),
    )(s1t, W1, b1, W2)


def _mm2_body(s2t_ref, b2_ref, wfc_ref, bfc_ref, out_ref, pred_ref):
    o = s2t_ref[...].T + b2_ref[...]
    out_ref[...] = o
    pred_ref[...] = jnp.dot(o, wfc_ref[...],
                            preferred_element_type=jnp.float32) + bfc_ref[...]


def _mm2(s2t, b2, Wfc, bfc):
    return pl.pallas_call(
        _mm2_body,
        out_shape=(jax.ShapeDtypeStruct((N, D), jnp.float32),
                   jax.ShapeDtypeStruct((N, D), jnp.float32)),
    )(s2t, b2, Wfc, bfc)


def kernel(x, edge_index, edge_attr, W1, b1, W2, b2, Wfc, bfc):
    row = edge_index[0]
    col = edge_index[1]

    part = _deg_partials(col, edge_attr)
    dinv2d, d22d, xt = _deg_finish(part, x)
    dinv = dinv2d.reshape(N)
    d2 = d22d.reshape(N)

    norm, pidx = _edge_norm(row, col, edge_attr, dinv)

    s1t = _agg_t(xt, pidx, norm, d2)
    x_emb, gt = _mm1(s1t, W1, b1.reshape(1, D_HID), W2)

    s2t = _agg_t(gt, pidx, norm, d2)
    out, predict = _mm2(s2t, b2.reshape(1, D), Wfc, bfc.reshape(1, D))

    return (out, x_emb, predict)


# speedup vs baseline: 1.0008x; 1.0008x over previous
import functools

import jax
import jax.numpy as jnp
from jax import lax
from jax.experimental import pallas as pl
from jax.experimental.pallas import tpu as pltpu
from jax.experimental.pallas import tpu_sc as plsc

N = 10000
E = 320000
D = 128
D_HID = 256

NC, NS, L = 2, 16, 16
NW = NC * NS
EPW = E // NW
CB = 3200
CBW = 2000
FPT = D // NW

_MESH = plsc.VectorSubcoreMesh(core_axis_name="c", subcore_axis_name="s",
                               num_cores=NC, num_subcores=NS)
_SC_PARAMS = pltpu.CompilerParams(needs_layout_passes=False)


def _wid():
    return lax.axis_index("s") * NC + lax.axis_index("c")


@functools.partial(
    pl.kernel,
    out_type=jax.ShapeDtypeStruct((NW, N), jnp.float32),
    mesh=_MESH,
    compiler_params=_SC_PARAMS,
    scratch_types=[
        pltpu.VMEM((N,), jnp.float32),
        pltpu.VMEM((EPW,), jnp.int32),
        pltpu.VMEM((EPW,), jnp.float32),
        pltpu.SemaphoreType.DMA,
    ],
)
def _deg_partials(col_hbm, ew_hbm, out_hbm, acc, colbuf, ewbuf, sem):
    wid = _wid()
    base = wid * EPW
    pltpu.async_copy(col_hbm.at[pl.ds(base, EPW)], colbuf, sem)
    pltpu.async_copy(ew_hbm.at[pl.ds(base, EPW)], ewbuf, sem)

    @plsc.parallel_loop(0, N // L, unroll=5)
    def zero_body(i):
        acc[pl.ds(i * L, L)] = jnp.zeros((L,), jnp.float32)

    pltpu.make_async_copy(col_hbm.at[pl.ds(0, EPW)], colbuf, sem).wait()
    pltpu.make_async_copy(ew_hbm.at[pl.ds(0, EPW)], ewbuf, sem).wait()

    @plsc.parallel_loop(0, EPW // L, unroll=5)
    def body(i):
        idx = colbuf[pl.ds(i * L, L)]
        w = ewbuf[pl.ds(i * L, L)]
        plsc.addupdate_scatter(acc, [idx], w)

    pltpu.sync_copy(acc, out_hbm.at[wid])


@functools.partial(
    pl.kernel,
    out_type=(jax.ShapeDtypeStruct((E,), jnp.float32),
              jax.ShapeDtypeStruct((E,), jnp.int32)),
    mesh=_MESH,
    compiler_params=_SC_PARAMS,
    scratch_types=[
        pltpu.VMEM((N,), jnp.float32),
        pltpu.VMEM((EPW,), jnp.int32),
        pltpu.VMEM((EPW,), jnp.int32),
        pltpu.VMEM((EPW,), jnp.float32),
        pltpu.VMEM((EPW,), jnp.float32),
        pltpu.VMEM((EPW,), jnp.int32),
        pltpu.SemaphoreType.DMA,
    ],
)
def _edge_norm(row_hbm, col_hbm, ew_hbm, dinv_hbm, out_hbm, pidx_hbm,
               dinvbuf, rowbuf, colbuf, ewbuf, normbuf, pidxbuf, sem):
    wid = _wid()
    base = wid * EPW
    pltpu.async_copy(row_hbm.at[pl.ds(base, EPW)], rowbuf, sem)
    pltpu.async_copy(col_hbm.at[pl.ds(base, EPW)], colbuf, sem)
    pltpu.async_copy(ew_hbm.at[pl.ds(base, EPW)], ewbuf, sem)
    pltpu.sync_copy(dinv_hbm, dinvbuf)
    pltpu.make_async_copy(row_hbm.at[pl.ds(0, EPW)], rowbuf, sem).wait()
    pltpu.make_async_copy(col_hbm.at[pl.ds(0, EPW)], colbuf, sem).wait()
    pltpu.make_async_copy(ew_hbm.at[pl.ds(0, EPW)], ewbuf, sem).wait()

    @plsc.parallel_loop(0, EPW // L, unroll=5)
    def body(i):
        s = pl.ds(i * L, L)
        r = rowbuf[s]
        cc = colbuf[s]
        w = ewbuf[s]
        dr = plsc.load_gather(dinvbuf, [r])
        dc = plsc.load_gather(dinvbuf, [cc])
        normbuf[s] = dr * w * dc
        pidxbuf[s] = jnp.bitwise_or(jnp.left_shift(r, 16), cc)

    pltpu.sync_copy(normbuf, out_hbm.at[pl.ds(base, EPW)])
    pltpu.sync_copy(pidxbuf, pidx_hbm.at[pl.ds(base, EPW)])


@functools.partial(
    pl.kernel,
    out_type=jax.ShapeDtypeStruct((D, N), jnp.float32),
    mesh=_MESH,
    compiler_params=_SC_PARAMS,
    scratch_types=[
        pltpu.VMEM((FPT, N), jnp.float32),
        pltpu.VMEM((FPT, N), jnp.float32),
        pltpu.VMEM((N,), jnp.float32),
        pltpu.VMEM((CB,), jnp.int32),
        pltpu.VMEM((CB,), jnp.int32),
        pltpu.VMEM((CB,), jnp.float32),
        pltpu.VMEM((CB,), jnp.float32),
        pltpu.SemaphoreType.DMA,
        pltpu.SemaphoreType.DMA,
    ],
)
def _agg_t(vt_hbm, pidx_hbm, norm_hbm, d2_hbm, out_hbm,
           vt, acc, d2buf, pidxbuf0, pidxbuf1, normbuf0, normbuf1,
           sem0, sem1):
    wid = _wid()
    f0 = wid * FPT
    NB = E // CB
    sems = (sem0, sem1)
    pidxbufs = (pidxbuf0, pidxbuf1)
    normbufs = (normbuf0, normbuf1)

    def issue(blk, slot):
        off = pl.multiple_of(blk * CB, 8)
        pltpu.async_copy(pidx_hbm.at[pl.ds(off, CB)], pidxbufs[slot], sems[slot])
        pltpu.async_copy(norm_hbm.at[pl.ds(off, CB)], normbufs[slot], sems[slot])

    def drain(slot):
        pltpu.make_async_copy(pidx_hbm.at[pl.ds(0, CB)], pidxbufs[slot],
                              sems[slot]).wait()
        pltpu.make_async_copy(norm_hbm.at[pl.ds(0, CB)], normbufs[slot],
                              sems[slot]).wait()

    issue(0, 0)
    issue(1, 1)

    pltpu.sync_copy(vt_hbm.at[pl.ds(f0, FPT)], vt)
    pltpu.sync_copy(d2_hbm, d2buf)

    for f in range(FPT):
        @plsc.parallel_loop(0, N // L, unroll=5)
        def init_body(i, f=f):
            s = pl.ds(i * L, L)
            acc[f, s] = vt[f, s] * d2buf[s]

    fidx = [jnp.full((L,), f, jnp.int32) for f in range(FPT)]

    def blk_body(k, c):
        for slot in range(2):
            blk = 2 * k + slot
            drain(slot)

            @plsc.parallel_loop(0, CB // L, unroll=8)
            def body(i, slot=slot):
                s = pl.ds(i * L, L)
                pv = pidxbufs[slot][s]
                r = jnp.right_shift(pv, 16)
                cc = jnp.bitwise_and(pv, jnp.int32(0xFFFF))
                nv = normbufs[slot][s]
                for f in range(FPT):
                    g = plsc.load_gather(vt, [fidx[f], r])
                    plsc.addupdate_scatter(acc, [fidx[f], cc], g * nv)

            @pl.when(blk + 2 < NB)
            def _(blk=blk, slot=slot):
                issue(blk + 2, slot)
        return c
    lax.fori_loop(0, NB // 2, blk_body, 0)

    pltpu.sync_copy(acc, out_hbm.at[pl.ds(f0, FPT)])


def _deg_finish_body(part_ref, x_ref, dinv_ref, d2_ref, xt_ref):
    deg = jnp.sum(part_ref[...], axis=0, keepdims=True) + 1.0
    dinv = jnp.where(deg > 0, lax.rsqrt(deg), 0.0)
    dinv_ref[...] = dinv
    d2_ref[...] = dinv * dinv
    xt_ref[...] = x_ref[...].T


def _deg_finish(part, x):
    return pl.pallas_call(
        _deg_finish_body,
        out_shape=(jax.ShapeDtypeStruct((1, N), jnp.float32),
                   jax.ShapeDtypeStruct((1, N), jnp.float32),
                   jax.ShapeDtypeStruct((D, N), jnp.float32)),
    )(part, x)


def _mm1_body(s1t_ref, w1_ref, b1_ref, w2_ref, xemb_ref, gt_ref):
    xe = lax.dot_general(s1t_ref[...], w1_ref[...],
                         (((0,), (0,)), ((), ())),
                         preferred_element_type=jnp.float32) + b1_ref[...]
    xemb_ref[...] = xe
    g = jnp.dot(jnp.maximum(xe, 0.0), w2_ref[...],
                preferred_element_type=jnp.float32)
    gt_ref[...] = g.T


def _mm1(s1t, W1, b1, W2):
    return pl.pallas_call(
        _mm1_body,
        out_shape=(jax.ShapeDtypeStruct((N, D_HID), jnp.float32),
                   jax.ShapeDtypeStruct((D, N), jnp.float32)),
    )(s1t, W1, b1, W2)


def _mm2_body(s2t_ref, b2_ref, wfc_ref, bfc_ref, out_ref, pred_ref):
    o = s2t_ref[...].T + b2_ref[...]
    out_ref[...] = o
    pred_ref[...] = jnp.dot(o, wfc_ref[...],
                            preferred_element_type=jnp.float32) + bfc_ref[...]


def _mm2(s2t, b2, Wfc, bfc):
    return pl.pallas_call(
        _mm2_body,
        out_shape=(jax.ShapeDtypeStruct((N, D), jnp.float32),
                   jax.ShapeDtypeStruct((N, D), jnp.float32)),
    )(s2t, b2, Wfc, bfc)


def kernel(x, edge_index, edge_attr, W1, b1, W2, b2, Wfc, bfc):
    row = edge_index[0]
    col = edge_index[1]

    part = _deg_partials(col, edge_attr)
    dinv2d, d22d, xt = _deg_finish(part, x)
    dinv = dinv2d.reshape(N)
    d2 = d22d.reshape(N)

    norm, pidx = _edge_norm(row, col, edge_attr, dinv)

    s1t = _agg_t(xt, pidx, norm, d2)
    x_emb, gt = _mm1(s1t, W1, b1.reshape(1, D_HID), W2)

    s2t = _agg_t(gt, pidx, norm, d2)
    out, predict = _mm2(s2t, b2.reshape(1, D), Wfc, bfc.reshape(1, D))

    return (out, x_emb, predict)
